# baseline (speedup 1.0000x reference)
import functools

import jax
import jax.numpy as jnp
from jax import lax
from jax.experimental import pallas as pl
from jax.experimental.pallas import tpu as pltpu
from jax.experimental.pallas import tpu_sc as plsc

N = 10000
E = 320000
F_IN = 128
HID = 64
C_OUT = 40
C_PAD = 64
DH = 32

NC = 2
NS = 16
NW = NC * NS
G = 128
NCHUNK = 80
NCHUNK2 = 160
EPT = G * NCHUNK
E_PAD = EPT * NW
SW = 624


def _sc_mesh():
    return plsc.VectorSubcoreMesh(core_axis_name="c", subcore_axis_name="s")


def _deg_partials(col3, w3):
    @functools.partial(
        pl.kernel,
        out_type=jax.ShapeDtypeStruct((NW, 1, N), jnp.float32),
        mesh=_sc_mesh(),
        compiler_params=pltpu.CompilerParams(needs_layout_passes=False),
        scratch_types=[
            pltpu.VMEM((NCHUNK, G), jnp.int32),
            pltpu.VMEM((NCHUNK, G), jnp.float32),
            pltpu.VMEM((1, N), jnp.float32),
        ],
    )
    def deg_kernel(col_hbm, w_hbm, out_hbm, col_v, w_v, deg_v):
        wid = lax.axis_index("s") * NC + lax.axis_index("c")
        pltpu.sync_copy(col_hbm.at[wid], col_v)
        pltpu.sync_copy(w_hbm.at[wid], w_v)

        zeros16 = jnp.zeros((16,), jnp.float32)
        zeros16i = jnp.zeros((16,), jnp.int32)

        def zbody(i, carry):
            deg_v[0, pl.ds(i * 16, 16)] = zeros16
            return carry

        lax.fori_loop(0, N // 16, zbody, 0)

        def ebody(i, carry):
            j = i // (G // 16)
            k = i % (G // 16)
            idx = col_v[j, pl.ds(k * 16, 16)]
            vals = w_v[j, pl.ds(k * 16, 16)]
            plsc.addupdate_scatter(deg_v, [zeros16i, idx], vals)
            return carry

        lax.fori_loop(0, NCHUNK * (G // 16), ebody, 0)
        pltpu.sync_copy(deg_v, out_hbm.at[wid])

    return deg_kernel(col3, w3)


def _edge_scatter(row3, col3, w3, y3):
    R, PF = 4, 2

    @functools.partial(
        pl.kernel,
        out_type=jax.ShapeDtypeStruct((NC, N, DH), jnp.float32),
        mesh=_sc_mesh(),
        compiler_params=pltpu.CompilerParams(needs_layout_passes=False,
                                             use_tc_tiling_on_sc=False),
        scratch_types=[
            pltpu.VMEM((NCHUNK2, G), jnp.int32),
            pltpu.VMEM((NCHUNK2, G), jnp.int32),
            pltpu.VMEM((NCHUNK2, G), jnp.float32),
            [pltpu.VMEM((G, DH), jnp.float32)] * R,
            pltpu.VMEM_SHARED((N, DH), jnp.float32),
            pltpu.VMEM_SHARED((N, DH), jnp.float32),
            [pltpu.SemaphoreType.DMA] * R,
            [pltpu.SemaphoreType.DMA] * R,
        ],
    )
    def sk(row_hbm, col_hbm, w_hbm, y_hbm, out_hbm,
           row_v, col_v, w_v, bufs, accum, ytab, gsems, ssems):
        cid = lax.axis_index("c")
        sid = lax.axis_index("s")
        pltpu.sync_copy(row_hbm.at[sid], row_v)
        pltpu.sync_copy(col_hbm.at[sid], col_v)
        pltpu.sync_copy(w_hbm.at[sid], w_v)

        zeros16 = jnp.zeros((16,), jnp.float32)
        nk = DH // 16
        b0 = bufs[0]

        def zb(i, carry):
            r = i // nk
            k = i % nk
            b0[r, pl.ds(k * 16, 16)] = zeros16
            return carry

        lax.fori_loop(0, G * nk, zb, 0)

        base = sid * SW

        def zs(t, carry):
            pltpu.sync_copy(b0, accum.at[pl.ds(base + t * G, G)])
            return carry

        lax.fori_loop(0, 640 // G, zs, 0)

        pltpu.sync_copy(y_hbm.at[pl.ds(base, SW), cid],
                        ytab.at[pl.ds(base, SW)])

        @pl.when(sid == NS - 1)
        def _stage_tail():
            pltpu.sync_copy(y_hbm.at[pl.ds(NS * SW, N - NS * SW), cid],
                            ytab.at[pl.ds(NS * SW, N - NS * SW)])

        plsc.subcore_barrier()

        def start_g(j, t):
            pltpu.async_copy(ytab.at[row_v.at[j]], bufs[t], gsems[t])

        def wait_g(t):
            pltpu.make_async_copy(ytab.at[row_v.at[0]], bufs[t],
                                  gsems[t]).wait()

        def start_s(j, t):
            pltpu.async_copy(bufs[t], accum.at[col_v.at[j]], ssems[t],
                             add=True)

        def wait_s(t):
            pltpu.make_async_copy(bufs[t], accum.at[col_v.at[0]],
                                  ssems[t]).wait()

        ivs = [jnp.full((16,), i, jnp.int32) for i in range(16)]

        def scale(j, t):
            b = bufs[t]

            @plsc.parallel_loop(0, G // 16, unroll=2)
            def grp(gi):
                wvec = w_v[j, pl.ds(gi * 16, 16)]
                for ii in range(16):
                    wb = wvec.at[ivs[ii]].get(mode="promise_in_bounds")
                    r = gi * 16 + ii
                    for k in range(nk):
                        sl = pl.ds(k * 16, 16)
                        b[r, sl] = b[r, sl] * wb

        for t in range(PF):
            start_g(t, t)
        for j in (0, 1):
            wait_g(j)
            start_g(j + PF, (j + PF) % R)
            scale(j, j)
            start_s(j, j)

        def main(kk, carry):
            jb = kk * R + 2
            for t in range(R):
                j = jb + t
                cur = (2 + t) % R
                nxt = t
                wait_g(cur)
                wait_s(nxt)
                start_g(j + PF, nxt)
                scale(j, cur)
                start_s(j, cur)
            return carry

        lax.fori_loop(0, (NCHUNK2 - PF - 2) // R, main, 0)
        for t in range(PF):
            j = NCHUNK2 - PF + t
            wait_g(j % R)
            scale(j, j % R)
            start_s(j, j % R)
        for t in range(R):
            wait_s(t)
        plsc.subcore_barrier()
        pltpu.sync_copy(accum.at[pl.ds(base, SW)],
                        out_hbm.at[cid, pl.ds(base, SW)])

        @pl.when(sid == NS - 1)
        def _tail():
            pltpu.sync_copy(accum.at[pl.ds(NS * SW, N - NS * SW)],
                            out_hbm.at[cid, pl.ds(NS * SW, N - NS * SW)])

    return sk(row3, col3, w3, y3)


def _tc_in_matmul(x, w0):
    def body(x_ref, w_ref, o_ref):
        o_ref[...] = jnp.dot(x_ref[...], w_ref[...],
                             preferred_element_type=jnp.float32)

    return pl.pallas_call(
        body, out_shape=jax.ShapeDtypeStruct((N, HID), jnp.float32))(x, w0)


def _tc_finalize_deg(dpt, h):
    def body(dp_ref, h_ref, dinv_ref, y_ref):
        ds = jnp.sum(dp_ref[...], axis=1, keepdims=True) + 1.0
        dinv = lax.rsqrt(ds)
        dinv_ref[...] = dinv
        y_ref[...] = h_ref[...] * dinv

    return pl.pallas_call(
        body,
        out_shape=(jax.ShapeDtypeStruct((N, 1), jnp.float32),
                   jax.ShapeDtypeStruct((N, HID), jnp.float32)))(dpt, h)


def _tc_mid(p, y1, dinv, b0r, w2p):
    def body(p_ref, y_ref, d_ref, b_ref, w_ref, o_ref):
        s = jnp.concatenate([p_ref[0], p_ref[1]], axis=1) + y_ref[...]
        h1 = jnp.maximum(d_ref[...] * s + b_ref[...], 0.0)
        g2 = jnp.dot(h1, w_ref[...], preferred_element_type=jnp.float32)
        o_ref[...] = d_ref[...] * g2

    return pl.pallas_call(
        body,
        out_shape=jax.ShapeDtypeStruct((N, C_PAD), jnp.float32))(
            p, y1, dinv, b0r, w2p)


def _tc_out(q, y2, dinv, b2r):
    def body(q_ref, y_ref, d_ref, b_ref, o_ref):
        z = d_ref[...] * (jnp.concatenate([q_ref[0], q_ref[1]], axis=1)
                          + y_ref[...])
        z40 = z[:, :C_OUT] + b_ref[...]
        m = jnp.max(z40, axis=1, keepdims=True)
        lse = jnp.log(jnp.sum(jnp.exp(z40 - m), axis=1, keepdims=True)) + m
        o_ref[...] = z40 - lse

    return pl.pallas_call(
        body,
        out_shape=jax.ShapeDtypeStruct((N, C_OUT), jnp.float32))(
            q, y2, dinv, b2r)


def kernel(x, edge_index, edge_attr, W0, b0, W2, b2):
    row = edge_index[0].astype(jnp.int32)
    col = edge_index[1].astype(jnp.int32)
    ew = edge_attr[:, 0].astype(jnp.float32)

    pad = E_PAD - E
    row_p = jnp.concatenate([row, jnp.zeros((pad,), jnp.int32)])
    col_p = jnp.concatenate([col, jnp.zeros((pad,), jnp.int32)])
    w_p = jnp.concatenate([ew, jnp.zeros((pad,), jnp.float32)])
    col3 = col_p.reshape(NW, NCHUNK, G)
    w3 = w_p.reshape(NW, NCHUNK, G)
    row16 = row_p.reshape(NS, NCHUNK2, G)
    col16 = col_p.reshape(NS, NCHUNK2, G)
    w16 = w_p.reshape(NS, NCHUNK2, G)

    dp = _deg_partials(col3, w3)
    h = _tc_in_matmul(x, W0)
    dinv, y1 = _tc_finalize_deg(dp.reshape(NW, N).T, h)
    p = _edge_scatter(row16, col16, w16, y1.reshape(N, NC, DH))
    w2p = jnp.zeros((HID, C_PAD), jnp.float32).at[:, :C_OUT].set(W2)
    y2 = _tc_mid(p, y1, dinv, b0.reshape(1, HID), w2p)
    q = _edge_scatter(row16, col16, w16, y2.reshape(N, NC, DH))
    return _tc_out(q, y2, dinv, b2.reshape(1, C_OUT))


# speedup vs baseline: 1.0054x; 1.0054x over previous
import functools

import jax
import jax.numpy as jnp
from jax import lax
from jax.experimental import pallas as pl
from jax.experimental.pallas import tpu as pltpu
from jax.experimental.pallas import tpu_sc as plsc

N = 10000
E = 320000
F_IN = 128
HID = 64
C_OUT = 40
C_PAD = 64
DH = 32

NC = 2
NS = 16
NW = NC * NS
G = 128
NCHUNK = 80
NCHUNK2 = 160
EPT = G * NCHUNK
E_PAD = EPT * NW
SW = 624


def _sc_mesh():
    return plsc.VectorSubcoreMesh(core_axis_name="c", subcore_axis_name="s")


def _deg_partials(col3, w3):
    @functools.partial(
        pl.kernel,
        out_type=jax.ShapeDtypeStruct((NW, 1, N), jnp.float32),
        mesh=_sc_mesh(),
        compiler_params=pltpu.CompilerParams(needs_layout_passes=False),
        scratch_types=[
            pltpu.VMEM((NCHUNK, G), jnp.int32),
            pltpu.VMEM((NCHUNK, G), jnp.float32),
            pltpu.VMEM((1, N), jnp.float32),
        ],
    )
    def deg_kernel(col_hbm, w_hbm, out_hbm, col_v, w_v, deg_v):
        wid = lax.axis_index("s") * NC + lax.axis_index("c")
        pltpu.sync_copy(col_hbm.at[wid], col_v)
        pltpu.sync_copy(w_hbm.at[wid], w_v)

        zeros16 = jnp.zeros((16,), jnp.float32)
        zeros16i = jnp.zeros((16,), jnp.int32)

        def zbody(i, carry):
            deg_v[0, pl.ds(i * 16, 16)] = zeros16
            return carry

        lax.fori_loop(0, N // 16, zbody, 0)

        def ebody(i, carry):
            j = i // (G // 16)
            k = i % (G // 16)
            idx = col_v[j, pl.ds(k * 16, 16)]
            vals = w_v[j, pl.ds(k * 16, 16)]
            plsc.addupdate_scatter(deg_v, [zeros16i, idx], vals)
            return carry

        lax.fori_loop(0, NCHUNK * (G // 16), ebody, 0)
        pltpu.sync_copy(deg_v, out_hbm.at[wid])

    return deg_kernel(col3, w3)


def _edge_scatter(row3, col3, w3, y3):
    R, PF = 4, 2

    @functools.partial(
        pl.kernel,
        out_type=jax.ShapeDtypeStruct((NC, N, DH), jnp.float32),
        mesh=_sc_mesh(),
        compiler_params=pltpu.CompilerParams(needs_layout_passes=False,
                                             use_tc_tiling_on_sc=False),
        scratch_types=[
            pltpu.VMEM((NCHUNK2, G), jnp.int32),
            pltpu.VMEM((NCHUNK2, G), jnp.int32),
            pltpu.VMEM((NCHUNK2, G), jnp.float32),
            [pltpu.VMEM((G, DH), jnp.float32)] * R,
            pltpu.VMEM_SHARED((N, DH), jnp.float32),
            pltpu.VMEM_SHARED((N, DH), jnp.float32),
            [pltpu.SemaphoreType.DMA] * R,
            [pltpu.SemaphoreType.DMA] * R,
        ],
    )
    def sk(row_hbm, col_hbm, w_hbm, y_hbm, out_hbm,
           row_v, col_v, w_v, bufs, accum, ytab, gsems, ssems):
        cid = lax.axis_index("c")
        sid = lax.axis_index("s")
        pltpu.sync_copy(row_hbm.at[sid], row_v)
        pltpu.sync_copy(col_hbm.at[sid], col_v)
        pltpu.sync_copy(w_hbm.at[sid], w_v)

        zeros16 = jnp.zeros((16,), jnp.float32)
        nk = DH // 16
        b0 = bufs[0]

        def zb(i, carry):
            r = i // nk
            k = i % nk
            b0[r, pl.ds(k * 16, 16)] = zeros16
            return carry

        lax.fori_loop(0, G * nk, zb, 0)

        base = sid * SW

        def zs(t, carry):
            pltpu.sync_copy(b0, accum.at[pl.ds(base + t * G, G)])
            return carry

        lax.fori_loop(0, 640 // G, zs, 0)

        pltpu.sync_copy(y_hbm.at[pl.ds(base, SW), cid],
                        ytab.at[pl.ds(base, SW)])

        @pl.when(sid == NS - 1)
        def _stage_tail():
            pltpu.sync_copy(y_hbm.at[pl.ds(NS * SW, N - NS * SW), cid],
                            ytab.at[pl.ds(NS * SW, N - NS * SW)])

        plsc.subcore_barrier()

        def start_g(j, t):
            pltpu.async_copy(ytab.at[row_v.at[j]], bufs[t], gsems[t])

        def wait_g(t):
            pltpu.make_async_copy(ytab.at[row_v.at[0]], bufs[t],
                                  gsems[t]).wait()

        def start_s(j, t):
            pltpu.async_copy(bufs[t], accum.at[col_v.at[j]], ssems[t],
                             add=True)

        def wait_s(t):
            pltpu.make_async_copy(bufs[t], accum.at[col_v.at[0]],
                                  ssems[t]).wait()

        def scale(j, t):
            b = bufs[t]
            jv = jnp.full((16,), j, jnp.int32)

            @plsc.parallel_loop(0, G, unroll=8)
            def rb(i):
                wv = plsc.load_gather(w_v, [jv, jnp.full((16,), i, jnp.int32)])
                for k in range(nk):
                    sl = pl.ds(k * 16, 16)
                    b[i, sl] = b[i, sl] * wv

        for t in range(PF):
            start_g(t, t)
        for j in (0, 1):
            wait_g(j)
            start_g(j + PF, (j + PF) % R)
            scale(j, j)
            start_s(j, j)

        def main(kk, carry):
            jb = kk * R + 2
            for t in range(R):
                j = jb + t
                cur = (2 + t) % R
                nxt = t
                wait_g(cur)
                wait_s(nxt)
                start_g(j + PF, nxt)
                scale(j, cur)
                start_s(j, cur)
            return carry

        lax.fori_loop(0, (NCHUNK2 - PF - 2) // R, main, 0)
        for t in range(PF):
            j = NCHUNK2 - PF + t
            wait_g(j % R)
            scale(j, j % R)
            start_s(j, j % R)
        for t in range(R):
            wait_s(t)
        plsc.subcore_barrier()
        pltpu.sync_copy(accum.at[pl.ds(base, SW)],
                        out_hbm.at[cid, pl.ds(base, SW)])

        @pl.when(sid == NS - 1)
        def _tail():
            pltpu.sync_copy(accum.at[pl.ds(NS * SW, N - NS * SW)],
                            out_hbm.at[cid, pl.ds(NS * SW, N - NS * SW)])

    return sk(row3, col3, w3, y3)


def _tc_in_matmul(x, w0):
    def body(x_ref, w_ref, o_ref):
        o_ref[...] = jnp.dot(x_ref[...], w_ref[...],
                             preferred_element_type=jnp.float32)

    return pl.pallas_call(
        body, out_shape=jax.ShapeDtypeStruct((N, HID), jnp.float32))(x, w0)


def _tc_finalize_deg(dpt, h):
    def body(dp_ref, h_ref, dinv_ref, y_ref):
        ds = jnp.sum(dp_ref[...], axis=1, keepdims=True) + 1.0
        dinv = lax.rsqrt(ds)
        dinv_ref[...] = dinv
        y_ref[...] = h_ref[...] * dinv

    return pl.pallas_call(
        body,
        out_shape=(jax.ShapeDtypeStruct((N, 1), jnp.float32),
                   jax.ShapeDtypeStruct((N, HID), jnp.float32)))(dpt, h)


def _tc_mid(p, y1, dinv, b0r, w2p):
    def body(p_ref, y_ref, d_ref, b_ref, w_ref, o_ref):
        s = jnp.concatenate([p_ref[0], p_ref[1]], axis=1) + y_ref[...]
        h1 = jnp.maximum(d_ref[...] * s + b_ref[...], 0.0)
        g2 = jnp.dot(h1, w_ref[...], preferred_element_type=jnp.float32)
        o_ref[...] = d_ref[...] * g2

    return pl.pallas_call(
        body,
        out_shape=jax.ShapeDtypeStruct((N, C_PAD), jnp.float32))(
            p, y1, dinv, b0r, w2p)


def _tc_out(q, y2, dinv, b2r):
    def body(q_ref, y_ref, d_ref, b_ref, o_ref):
        z = d_ref[...] * (jnp.concatenate([q_ref[0], q_ref[1]], axis=1)
                          + y_ref[...])
        z40 = z[:, :C_OUT] + b_ref[...]
        m = jnp.max(z40, axis=1, keepdims=True)
        lse = jnp.log(jnp.sum(jnp.exp(z40 - m), axis=1, keepdims=True)) + m
        o_ref[...] = z40 - lse

    return pl.pallas_call(
        body,
        out_shape=jax.ShapeDtypeStruct((N, C_OUT), jnp.float32))(
            q, y2, dinv, b2r)


def kernel(x, edge_index, edge_attr, W0, b0, W2, b2):
    row = edge_index[0].astype(jnp.int32)
    col = edge_index[1].astype(jnp.int32)
    ew = edge_attr[:, 0].astype(jnp.float32)

    pad = E_PAD - E
    row_p = jnp.concatenate([row, jnp.zeros((pad,), jnp.int32)])
    col_p = jnp.concatenate([col, jnp.zeros((pad,), jnp.int32)])
    w_p = jnp.concatenate([ew, jnp.zeros((pad,), jnp.float32)])
    col3 = col_p.reshape(NW, NCHUNK, G)
    w3 = w_p.reshape(NW, NCHUNK, G)
    row16 = row_p.reshape(NS, NCHUNK2, G)
    col16 = col_p.reshape(NS, NCHUNK2, G)
    w16 = w_p.reshape(NS, NCHUNK2, G)

    dp = _deg_partials(col3, w3)
    h = _tc_in_matmul(x, W0)
    dinv, y1 = _tc_finalize_deg(dp.reshape(NW, N).T, h)
    p = _edge_scatter(row16, col16, w16, y1.reshape(N, NC, DH))
    w2p = jnp.zeros((HID, C_PAD), jnp.float32).at[:, :C_OUT].set(W2)
    y2 = _tc_mid(p, y1, dinv, b0.reshape(1, HID), w2p)
    q = _edge_scatter(row16, col16, w16, y2.reshape(N, NC, DH))
    return _tc_out(q, y2, dinv, b2.reshape(1, C_OUT))
